# baseline (speedup 1.0000x reference)
import functools

import jax
import jax.numpy as jnp
from jax import lax
from jax.experimental import pallas as pl
from jax.experimental.pallas import tpu as pltpu
from jax.experimental.pallas import tpu_sc as plsc

N_TOKENS = 1000
BATCH = 16384
SEQ_LEN = 200
OUT_COLS = N_TOKENS - 1

_INFO = plsc.get_sparse_core_info()
NUM_CORES = _INFO.num_cores
NUM_SUBCORES = _INFO.num_subcores
LANES = _INFO.num_lanes
NW = NUM_CORES * NUM_SUBCORES

RPW = BATCH // NW
CHUNK = 32
NCHUNK = RPW // CHUNK
ZFULL = OUT_COLS // LANES
ZTAIL = OUT_COLS - LANES
SLAB = 8
NSLAB = CHUNK // SLAB
FULL_GROUPS = SEQ_LEN // LANES
TAIL_OFF = SEQ_LEN - LANES
OUT_TILES = OUT_COLS // 128
OUT_REM = OUT_COLS - OUT_TILES * 128


def _bow_body(in_hbm, out_hbm, sa, sb, cnt0, cnt1,
              in_sem_a, in_sem_b, out_sem0, out_sem1):
    wid = lax.axis_index("s") * NUM_CORES + lax.axis_index("c")
    iota = lax.iota(jnp.int32, LANES)
    zeros = jnp.zeros((LANES,), jnp.float32)
    ones = jnp.ones((LANES,), jnp.float32)
    tail_mask = iota >= (LANES - (SEQ_LEN - FULL_GROUPS * LANES))

    cnts = (cnt0, cnt1)
    out_sems = (out_sem0, out_sem1)
    base = wid * RPW

    def start_slab(k, j, buf, sem):
        row = base + k * CHUNK + j * SLAB
        pltpu.async_copy(in_hbm.at[pl.ds(row, SLAB), pl.ds(0, 128)],
                         buf.at[:, pl.ds(0, 128)], sem)
        pltpu.async_copy(in_hbm.at[pl.ds(row, SLAB), pl.ds(128, SEQ_LEN - 128)],
                         buf.at[:, pl.ds(128, SEQ_LEN - 128)], sem)

    def wait_slab(buf, sem):
        pltpu.make_async_copy(in_hbm.at[pl.ds(0, SLAB), pl.ds(0, 128)],
                              buf.at[:, pl.ds(0, 128)], sem).wait()
        pltpu.make_async_copy(in_hbm.at[pl.ds(0, SLAB), pl.ds(128, SEQ_LEN - 128)],
                              buf.at[:, pl.ds(128, SEQ_LEN - 128)], sem).wait()

    def make_zero(cnt):
        def zero_row(r):
            for c in range(ZFULL):
                cnt[r, pl.ds(c * LANES, LANES)] = zeros
            plsc.store_scatter(cnt, [jnp.full((LANES,), r, jnp.int32),
                                     iota + ZTAIL], zeros)
        return zero_row

    def make_scatter(slab, cnt, j):
        def row_body(r):
            rowvec = jnp.full((LANES,), j * SLAB + r, jnp.int32)
            for c in range(FULL_GROUPS):
                toks = slab[r, pl.ds(c * LANES, LANES)]
                plsc.addupdate_scatter(cnt, [rowvec, toks - 1], ones,
                                       mask=toks != 0)
            toks = slab[r, pl.ds(TAIL_OFF, LANES)]
            plsc.addupdate_scatter(cnt, [rowvec, toks - 1], ones,
                                   mask=tail_mask & (toks != 0))
        return row_body

    def enqueue_drain(k, p):
        row = base + k * CHUNK
        pltpu.async_copy(cnts[p], out_hbm.at[pl.ds(row, CHUNK), :], out_sems[p])

    def wait_drain(p):
        pltpu.make_async_copy(cnts[p], out_hbm.at[pl.ds(0, CHUNK), :],
                              out_sems[p]).wait()

    def run_chunk(k, p):
        cnt = cnts[p]

        @pl.when(k >= 2)
        def _():
            wait_drain(p)
        plsc.parallel_loop(0, CHUNK, unroll=2)(make_zero(cnt))

        start_slab(k, 1, sb, in_sem_b)
        wait_slab(sa, in_sem_a)
        plsc.parallel_loop(0, SLAB, unroll=2)(make_scatter(sa, cnt, 0))
        start_slab(k, 2, sa, in_sem_a)
        wait_slab(sb, in_sem_b)
        plsc.parallel_loop(0, SLAB, unroll=2)(make_scatter(sb, cnt, 1))
        start_slab(k, 3, sb, in_sem_b)
        wait_slab(sa, in_sem_a)
        plsc.parallel_loop(0, SLAB, unroll=2)(make_scatter(sa, cnt, 2))

        @pl.when(k < NCHUNK - 1)
        def _():
            start_slab(k + 1, 0, sa, in_sem_a)
        wait_slab(sb, in_sem_b)
        plsc.parallel_loop(0, SLAB, unroll=2)(make_scatter(sb, cnt, 3))

        enqueue_drain(k, p)

    start_slab(0, 0, sa, in_sem_a)

    @pl.loop(0, NCHUNK // 2)
    def chunk_pair(i):
        run_chunk(i * 2, 0)
        run_chunk(i * 2 + 1, 1)

    wait_drain(0)
    wait_drain(1)


_bow_kernel = functools.partial(
    pl.kernel,
    out_type=jax.ShapeDtypeStruct((BATCH, OUT_COLS), jnp.float32),
    mesh=plsc.VectorSubcoreMesh(core_axis_name="c", subcore_axis_name="s"),
    scratch_types=[
        pltpu.VMEM((SLAB, SEQ_LEN), jnp.int32),
        pltpu.VMEM((SLAB, SEQ_LEN), jnp.int32),
        pltpu.VMEM((CHUNK, OUT_COLS), jnp.float32),
        pltpu.VMEM((CHUNK, OUT_COLS), jnp.float32),
        pltpu.SemaphoreType.DMA,
        pltpu.SemaphoreType.DMA,
        pltpu.SemaphoreType.DMA,
        pltpu.SemaphoreType.DMA,
    ],
    compiler_params=pltpu.CompilerParams(
        needs_layout_passes=False,
        use_tc_tiling_on_sc=True,
    ),
)(_bow_body)


@jax.jit
def kernel(inputs):
    return _bow_kernel(inputs)


# speedup vs baseline: 1.2532x; 1.2532x over previous
import functools

import jax
import jax.numpy as jnp
from jax import lax
from jax.experimental import pallas as pl
from jax.experimental.pallas import tpu as pltpu
from jax.experimental.pallas import tpu_sc as plsc

N_TOKENS = 1000
BATCH = 16384
SEQ_LEN = 200
OUT_COLS = N_TOKENS - 1

_INFO = plsc.get_sparse_core_info()
NUM_CORES = _INFO.num_cores
NUM_SUBCORES = _INFO.num_subcores
LANES = _INFO.num_lanes
NW = NUM_CORES * NUM_SUBCORES

STRIPE = 128
SPW = BATCH // (NW * STRIPE)
SEQ_TILES = SEQ_LEN // 8
HALF0 = 496
HALF1 = OUT_COLS - HALF0
TILE_GROUPS = 8 * STRIPE // LANES


def _bow_body(in_hbm, out_hbm, ina, inb, cnt0, cnt1,
              in_sem_a, in_sem_b, out_sem0, out_sem1):
    wid = lax.axis_index("s") * NUM_CORES + lax.axis_index("c")
    iota = lax.iota(jnp.int32, LANES)
    zeros = jnp.zeros((LANES,), jnp.float32)
    ones = jnp.ones((LANES,), jnp.float32)

    cnts = (cnt0, cnt1)
    out_sems = (out_sem0, out_sem1)
    halves = ((0, HALF0), (HALF0, HALF1))
    base = wid * SPW * STRIPE

    cols = [k + iota for k in range(0, STRIPE, LANES)]
    nrows_u = {}

    def make_zero(cnt):
        def zero_row(r):
            for u in range(STRIPE // LANES):
                cnt[r, pl.ds(u * LANES, LANES)] = zeros
        return zero_row

    def make_scatter(inb_, cnt, r0, nrows):
        lo = r0 + 1
        bound = nrows_u.setdefault(nrows, jnp.uint32(nrows))

        def tile_row(i):
            for u in range(STRIPE // LANES):
                x = inb_[i, pl.ds(u * LANES, LANES)] - lo
                plsc.addupdate_scatter(
                    cnt, [x, cols[u]], ones, mask=x.astype(jnp.uint32) < bound)
        return tile_row

    out_cp = {}
    for u in range(SPW * 2):
        s, h = u // 2, u % 2
        r0, nrows = halves[h]
        cnt = cnts[h]
        col = pl.ds(base + s * STRIPE, STRIPE)

        def start(t, buf, sem):
            return pltpu.async_copy(
                in_hbm.at[pl.ds(t * 8, 8), col], buf, sem)

        def wait(buf, sem):
            pltpu.make_async_copy(
                in_hbm.at[pl.ds(0, 8), col], buf, sem).wait()

        if u >= 2:
            out_cp[u - 2].wait()
        plsc.parallel_loop(0, nrows * 8, unroll=12)(make_zero(cnt))

        scat_a = make_scatter(ina, cnt, r0, nrows)
        scat_b = make_scatter(inb, cnt, r0, nrows)
        start(0, ina, in_sem_a)

        @pl.loop(0, SEQ_TILES // 2)
        def tile_pair(i):
            t = i * 2
            start(t + 1, inb, in_sem_b)
            wait(ina, in_sem_a)
            plsc.parallel_loop(0, 8, unroll=2)(scat_a)
            start(t + 2, ina, in_sem_a)
            wait(inb, in_sem_b)
            plsc.parallel_loop(0, 8, unroll=2)(scat_b)

        wait(ina, in_sem_a)
        plsc.parallel_loop(0, 8, unroll=2)(scat_a)

        out_cp[u] = pltpu.async_copy(
            cnt, out_hbm.at[pl.ds(r0, nrows), col], out_sems[h])
    out_cp[SPW * 2 - 2].wait()
    out_cp[SPW * 2 - 1].wait()


_bow_kernel = functools.partial(
    pl.kernel,
    out_type=jax.ShapeDtypeStruct((OUT_COLS, BATCH), jnp.float32),
    mesh=plsc.VectorSubcoreMesh(core_axis_name="c", subcore_axis_name="s"),
    scratch_types=[
        pltpu.VMEM((8, STRIPE), jnp.int32),
        pltpu.VMEM((8, STRIPE), jnp.int32),
        pltpu.VMEM((HALF0, STRIPE), jnp.float32),
        pltpu.VMEM((HALF1, STRIPE), jnp.float32),
        pltpu.SemaphoreType.DMA,
        pltpu.SemaphoreType.DMA,
        pltpu.SemaphoreType.DMA,
        pltpu.SemaphoreType.DMA,
    ],
    compiler_params=pltpu.CompilerParams(
        needs_layout_passes=False,
        use_tc_tiling_on_sc=True,
    ),
)(_bow_body)


@jax.jit
def kernel(inputs):
    out_t = _bow_kernel(inputs.T)
    return out_t.T


# speedup vs baseline: 1.3546x; 1.0809x over previous
import functools

import jax
import jax.numpy as jnp
from jax import lax
from jax.experimental import pallas as pl
from jax.experimental.pallas import tpu as pltpu
from jax.experimental.pallas import tpu_sc as plsc

N_TOKENS = 1000
BATCH = 16384
SEQ_LEN = 200
OUT_COLS = N_TOKENS - 1

_INFO = plsc.get_sparse_core_info()
NUM_CORES = _INFO.num_cores
NUM_SUBCORES = _INFO.num_subcores
LANES = _INFO.num_lanes
NW = NUM_CORES * NUM_SUBCORES

STRIPE = 128
SPW = BATCH // (NW * STRIPE)
SEQ_TILES = SEQ_LEN // 8
HALF0 = 496
HALF1 = OUT_COLS - HALF0
TILE_GROUPS = 8 * STRIPE // LANES


def _bow_body(in_hbm, out_hbm, ina, inb, cnt0, cnt1,
              in_sem_a, in_sem_b, out_sem0, out_sem1):
    wid = lax.axis_index("s") * NUM_CORES + lax.axis_index("c")
    iota = lax.iota(jnp.int32, LANES)
    zeros = jnp.zeros((LANES,), jnp.float32)
    ones = jnp.ones((LANES,), jnp.float32)

    cnts = (cnt0, cnt1)
    out_sems = (out_sem0, out_sem1)
    halves = ((0, HALF0), (HALF0, HALF1))
    base = wid * SPW * STRIPE

    def make_zero(cnt):
        def zero_step(j):
            cnt[j >> 3, pl.ds((j & 7) * LANES, LANES)] = zeros
        return zero_step

    def make_scatter(inb_, cnt, r0, nrows):
        lo = r0 + 1
        bound = jnp.uint32(nrows)

        def tok_step(g):
            k = (g & 7) * LANES
            x = inb_[g >> 3, pl.ds(k, LANES)] - lo
            plsc.addupdate_scatter(
                cnt, [x, k + iota], ones, mask=x.astype(jnp.uint32) < bound)
        return tok_step

    out_cp = {}
    for u in range(SPW * 2):
        s, h = u // 2, u % 2
        r0, nrows = halves[h]
        cnt = cnts[h]
        col = pl.ds(base + s * STRIPE, STRIPE)

        def start(t, buf, sem):
            return pltpu.async_copy(
                in_hbm.at[pl.ds(t * 8, 8), col], buf, sem)

        def wait(buf, sem):
            pltpu.make_async_copy(
                in_hbm.at[pl.ds(0, 8), col], buf, sem).wait()

        if u >= 2:
            out_cp[u - 2].wait()
        plsc.parallel_loop(0, nrows * 8, unroll=12)(make_zero(cnt))

        scat_a = make_scatter(ina, cnt, r0, nrows)
        scat_b = make_scatter(inb, cnt, r0, nrows)
        start(0, ina, in_sem_a)

        @pl.loop(0, SEQ_TILES // 2)
        def tile_pair(i):
            t = i * 2
            start(t + 1, inb, in_sem_b)
            wait(ina, in_sem_a)
            plsc.parallel_loop(0, TILE_GROUPS, unroll=8)(scat_a)
            start(t + 2, ina, in_sem_a)
            wait(inb, in_sem_b)
            plsc.parallel_loop(0, TILE_GROUPS, unroll=8)(scat_b)

        wait(ina, in_sem_a)
        plsc.parallel_loop(0, TILE_GROUPS, unroll=8)(scat_a)

        out_cp[u] = pltpu.async_copy(
            cnt, out_hbm.at[pl.ds(r0, nrows), col], out_sems[h])
    out_cp[SPW * 2 - 2].wait()
    out_cp[SPW * 2 - 1].wait()


_bow_kernel = functools.partial(
    pl.kernel,
    out_type=jax.ShapeDtypeStruct((OUT_COLS, BATCH), jnp.float32),
    mesh=plsc.VectorSubcoreMesh(core_axis_name="c", subcore_axis_name="s"),
    scratch_types=[
        pltpu.VMEM((8, STRIPE), jnp.int32),
        pltpu.VMEM((8, STRIPE), jnp.int32),
        pltpu.VMEM((HALF0, STRIPE), jnp.float32),
        pltpu.VMEM((HALF1, STRIPE), jnp.float32),
        pltpu.SemaphoreType.DMA,
        pltpu.SemaphoreType.DMA,
        pltpu.SemaphoreType.DMA,
        pltpu.SemaphoreType.DMA,
    ],
    compiler_params=pltpu.CompilerParams(
        needs_layout_passes=False,
        use_tc_tiling_on_sc=True,
    ),
)(_bow_body)


@jax.jit
def kernel(inputs):
    out_t = _bow_kernel(inputs.T)
    return out_t.T
